# baseline (speedup 1.0000x reference)
import functools

import jax
import jax.numpy as jnp
from jax import lax
from jax.experimental import pallas as pl
from jax.experimental.pallas import tpu as pltpu
from jax.experimental.pallas import tpu_sc as plsc

BATCH = 16384
EMBED_DIM = 64
LANES = 16

_info = plsc.get_sparse_core_info()
_NC = _info.num_cores
_NS = _info.num_subcores
NW = _NC * _NS
B_PER_W = BATCH // NW
CHUNK = 128
NCHUNK = B_PER_W // CHUNK
VPR = EMBED_DIM // LANES


def _gmf_body(uidx_hbm, iidx_hbm, utab_hbm, itab_hbm, out_hbm,
              uidx_v, iidx_v, urows_v, irows_v, sem):
    wid = lax.axis_index("s") * _NC + lax.axis_index("c")
    base = wid * B_PER_W
    crow = wid * NCHUNK

    pltpu.sync_copy(uidx_hbm.at[pl.ds(crow, NCHUNK)], uidx_v)
    pltpu.sync_copy(iidx_hbm.at[pl.ds(crow, NCHUNK)], iidx_v)

    copies = []
    for j in range(NCHUNK):
        copies.append(pltpu.async_copy(
            utab_hbm.at[uidx_v.at[j]],
            urows_v.at[pl.ds(j * CHUNK, CHUNK)], sem))
        copies.append(pltpu.async_copy(
            itab_hbm.at[iidx_v.at[j]],
            irows_v.at[pl.ds(j * CHUNK, CHUNK)], sem))
    for c in copies:
        c.wait()

    def mul_row(r, _):
        for c in range(VPR):
            sl = pl.ds(c * LANES, LANES)
            urows_v[r, sl] = urows_v[r, sl] * irows_v[r, sl]
        return _

    lax.fori_loop(0, B_PER_W, mul_row, None)

    pltpu.sync_copy(urows_v, out_hbm.at[pl.ds(base, B_PER_W)])


@jax.jit
def _gmf(uidx, iidx, utab, itab):
    mesh = plsc.VectorSubcoreMesh(core_axis_name="c", subcore_axis_name="s")
    kfn = functools.partial(
        pl.kernel,
        mesh=mesh,
        out_type=jax.ShapeDtypeStruct((BATCH, EMBED_DIM), jnp.float32),
        scratch_types=[
            pltpu.VMEM((NCHUNK, CHUNK), jnp.int32),
            pltpu.VMEM((NCHUNK, CHUNK), jnp.int32),
            pltpu.VMEM((B_PER_W, EMBED_DIM), jnp.float32),
            pltpu.VMEM((B_PER_W, EMBED_DIM), jnp.float32),
            pltpu.SemaphoreType.DMA,
        ],
        compiler_params=pltpu.CompilerParams(use_tc_tiling_on_sc=False),
    )(_gmf_body)
    return kfn(uidx, iidx, utab, itab)


def kernel(user_indices, item_indices, user_table, item_table):
    uidx = user_indices.astype(jnp.int32).reshape(BATCH // CHUNK, CHUNK)
    iidx = item_indices.astype(jnp.int32).reshape(BATCH // CHUNK, CHUNK)
    return _gmf(uidx, iidx, user_table, item_table)


# speedup vs baseline: 1.5737x; 1.5737x over previous
import functools

import jax
import jax.numpy as jnp
from jax import lax
from jax.experimental import pallas as pl
from jax.experimental.pallas import tpu as pltpu
from jax.experimental.pallas import tpu_sc as plsc

BATCH = 16384
EMBED_DIM = 64
LANES = 16

_info = plsc.get_sparse_core_info()
_NC = _info.num_cores
_NS = _info.num_subcores
NW = _NC * _NS
B_PER_W = BATCH // NW
HALF = B_PER_W // 2
NGROUP = HALF // LANES
VPR = EMBED_DIM // LANES


def _gmf_body(uidx_hbm, iidx_hbm, utab_hbm, itab_hbm, out_hbm,
              uidx_v, iidx_v, urows_v, irows_v, sem):
    wid = lax.axis_index("s") * _NC + lax.axis_index("c")
    base = wid * B_PER_W

    pltpu.sync_copy(uidx_hbm.at[pl.ds(base, B_PER_W)], uidx_v)
    pltpu.sync_copy(iidx_hbm.at[pl.ds(base, B_PER_W)], iidx_v)

    def half_body(h, carry):
        def fire_group(g, c2):
            uvec = uidx_v[pl.ds(h * HALF + g * LANES, LANES)]
            ivec = iidx_v[pl.ds(h * HALF + g * LANES, LANES)]
            for l in range(LANES):
                pltpu.async_copy(
                    utab_hbm.at[pl.ds(uvec[l], 1)],
                    urows_v.at[pl.ds(g * LANES + l, 1)], sem)
                pltpu.async_copy(
                    itab_hbm.at[pl.ds(ivec[l], 1)],
                    irows_v.at[pl.ds(g * LANES + l, 1)], sem)
            return c2

        lax.fori_loop(0, NGROUP, fire_group, None)

        pltpu.make_async_copy(utab_hbm.at[pl.ds(0, HALF)], urows_v, sem).wait()
        pltpu.make_async_copy(itab_hbm.at[pl.ds(0, HALF)], irows_v, sem).wait()

        def mul_row(i, c2):
            for c in range(VPR):
                sl = pl.ds(c * LANES, LANES)
                urows_v[i, sl] = urows_v[i, sl] * irows_v[i, sl]
            return c2

        lax.fori_loop(0, HALF, mul_row, None)

        pltpu.sync_copy(urows_v, out_hbm.at[pl.ds(base + h * HALF, HALF)])
        return carry

    lax.fori_loop(0, 2, half_body, None)


@jax.jit
def _gmf(uidx, iidx, utab, itab):
    mesh = plsc.VectorSubcoreMesh(core_axis_name="c", subcore_axis_name="s")
    kfn = functools.partial(
        pl.kernel,
        mesh=mesh,
        out_type=jax.ShapeDtypeStruct((BATCH, EMBED_DIM), jnp.float32),
        scratch_types=[
            pltpu.VMEM((B_PER_W,), jnp.int32),
            pltpu.VMEM((B_PER_W,), jnp.int32),
            pltpu.VMEM((HALF, EMBED_DIM), jnp.float32),
            pltpu.VMEM((HALF, EMBED_DIM), jnp.float32),
            pltpu.SemaphoreType.DMA,
        ],
    )(_gmf_body)
    return kfn(uidx, iidx, utab, itab)


def kernel(user_indices, item_indices, user_table, item_table):
    uidx = user_indices.astype(jnp.int32)
    iidx = item_indices.astype(jnp.int32)
    return _gmf(uidx, iidx, user_table, item_table)
